# Initial kernel scaffold; baseline (speedup 1.0000x reference)
import jax
import jax.numpy as jnp
from jax.experimental import pallas as pl


def kernel(x, pos, Wq, Wk, Wv, Wvoff, Woff1, boff1, ln_g, ln_b, Woff2, Wproj, bproj):
    raise NotImplementedError("write your pallas kernel here")


# speedup vs baseline: 8.1078x; 8.1078x over previous
import functools

import jax
import jax.numpy as jnp
from jax.experimental import pallas as pl
from jax.experimental.pallas import tpu as pltpu

B, N, DIM, HEADS, K, NGROUP = 8, 512, 384, 6, 10, 2
HD = DIM // HEADS
GD = DIM // NGROUP
TB = 128
NTB = N // TB
RB = K * TB

_F32 = jnp.float32
_I32 = jnp.int32
import numpy as _np
_SQRT_HALF = _np.sqrt(0.5).astype(_np.float32)


def _topk_small(D, k):
    R = D.shape[0]
    iota = jax.lax.broadcasted_iota(_I32, (R, N), 1)
    vals, idxs = [], []
    for _ in range(k):
        m = jnp.min(D, axis=1, keepdims=True)
        am = jnp.min(jnp.where(D == m, iota, N), axis=1, keepdims=True)
        vals.append(m)
        idxs.append(am)
        D = jnp.where(iota == am, jnp.inf, D)
    return vals, idxs


def _mmx(a, b):
    return jnp.dot(a.astype(jnp.bfloat16), b.astype(jnp.bfloat16),
                   preferred_element_type=_F32)


def _sqd(a, bT):
    b2 = jnp.sum(bT * bT, axis=0, keepdims=True)
    a2 = jnp.sum(a * a, axis=1, keepdims=True)
    ab = a.astype(jnp.bfloat16).astype(_F32)
    bTb = bT.astype(jnp.bfloat16).astype(_F32)
    dot = ab[:, 0:1] * bTb[0:1, :]
    dot = dot + ab[:, 1:2] * bTb[1:2, :]
    dot = dot + ab[:, 2:3] * bTb[2:3, :]
    return (-2.0 * dot + a2) + b2


def _knn_body(pos_ref, posT_ref, idx_ref):
    p = pos_ref[0]
    pT = posT_ref[0]
    D = _sqd(p, pT)
    _, idxs = _topk_small(D, K)
    idx_ref[0] = jnp.concatenate(idxs, axis=1).astype(_I32)


def _proj_body(x_ref, wqT_ref, wvT_ref, q_ref, v_ref):
    x = x_ref[...]
    q_ref[...] = _mmx(x, wqT_ref[...])
    v_ref[...] = _mmx(x, wvT_ref[...])


def _onehot_rows(idx2):
    iota = jax.lax.broadcasted_iota(_I32, (TB, N), 1)
    ohs = [(idx2[:, k:k + 1] == iota).astype(_F32) for k in range(K)]
    return jnp.stack(ohs, axis=0).reshape(RB, N)


def _gather_exact(oh, v):
    bf = jnp.bfloat16
    hi = v.astype(bf)
    r1 = v - hi.astype(_F32)
    mid = r1.astype(bf)
    lo = (r1 - mid.astype(_F32)).astype(bf)
    ohb = oh.astype(bf)
    out = jnp.dot(ohb, hi, preferred_element_type=_F32)
    out = out + jnp.dot(ohb, mid, preferred_element_type=_F32)
    out = out + jnp.dot(ohb, lo, preferred_element_type=_F32)
    return out


def _offmlp_body(idx_ref, q_ref, voff_ref, pos_ref, posT_ref, xg_ref,
                 w1T_ref, b1_ref, lng_ref, lnb_ref, w2T_ref, out_ref):
    oh = _onehot_rows(idx_ref[0])
    lv = _gather_exact(oh, voff_ref[0, 0])
    lpos = _gather_exact(oh, pos_ref[0])
    gq3 = jnp.broadcast_to(q_ref[0, 0][None], (K, TB, GD)).reshape(RB, GD)
    sf = jnp.concatenate([lv, gq3], axis=1)
    h = _mmx(sf, w1T_ref[...]) + b1_ref[0]
    mu = jnp.mean(h, axis=1, keepdims=True)
    var = jnp.mean(jax.lax.square(h - mu), axis=1, keepdims=True)
    h = (h - mu) / jnp.sqrt(var + 1e-5) * lng_ref[0] + lnb_ref[0]
    h = 0.5 * h * (1.0 - jax.lax.erf(-h * _SQRT_HALF))
    off = jnp.tanh(_mmx(h, w2T_ref[...]))
    sp = lpos + off
    D2 = _sqd(sp, posT_ref[0])
    vals, idxs = _topk_small(D2, 3)
    iota = jax.lax.broadcasted_iota(_I32, (RB, N), 1)
    recips = [1.0 / (v + 1e-8) for v in vals]
    s = recips[0] + recips[1] + recips[2]
    Wm = ((iota == idxs[0]).astype(_F32) * recips[0]
          + (iota == idxs[1]).astype(_F32) * recips[1]
          + (iota == idxs[2]).astype(_F32) * recips[2]) / s
    out_ref[0, 0] = jnp.dot(Wm, xg_ref[0, 0], preferred_element_type=_F32)


def _attn_body(interp_ref, idx_ref, q_ref, wkT_ref, wvT_ref,
               wpT_ref, bp_ref, y_ref):
    i4 = interp_ref[0]
    interp = jnp.concatenate([i4[0], i4[1]], axis=1)
    kh = _mmx(interp, wkT_ref[...])
    vh = _mmx(interp, wvT_ref[...])
    oh = _onehot_rows(idx_ref[0])
    lq = jnp.dot(oh, q_ref[0], preferred_element_type=_F32)
    q3 = lq.reshape(K, TB, DIM)
    k3 = kh.reshape(K, TB, DIM)
    v3 = vh.reshape(K, TB, DIM)
    scale = HD ** -0.5
    outs = []
    for h in range(HEADS):
        sl = slice(h * HD, (h + 1) * HD)
        qh = q3[:, :, sl]
        khh = k3[:, :, sl]
        vhh = v3[:, :, sl]
        lgs = []
        for n in range(K):
            lgs.append(jnp.sum(qh * khh[n][None], axis=-1) * scale)
        L = jnp.stack(lgs, axis=1)
        mx = jnp.max(L, axis=1, keepdims=True)
        e = jnp.exp(L - mx)
        attn = e / jnp.sum(e, axis=1, keepdims=True)
        acc = attn[:, 0, :][:, :, None] * vhh[0][None]
        for n in range(1, K):
            acc = acc + attn[:, n, :][:, :, None] * vhh[n][None]
        outs.append(jnp.max(acc, axis=0))
    om = jnp.concatenate(outs, axis=1)
    y_ref[0] = _mmx(om, wpT_ref[...]) + bp_ref[0]


@jax.jit
def kernel(x, pos, Wq, Wk, Wv, Wvoff, Woff1, boff1, ln_g, ln_b, Woff2,
           Wproj, bproj):
    posT = jnp.swapaxes(pos, 1, 2)
    cp = pltpu.CompilerParams(
        dimension_semantics=("parallel",))

    idx = pl.pallas_call(
        _knn_body,
        grid=(B,),
        in_specs=[
            pl.BlockSpec((1, N, 3), lambda b: (b, 0, 0)),
            pl.BlockSpec((1, 3, N), lambda b: (b, 0, 0)),
        ],
        out_specs=pl.BlockSpec((1, N, K), lambda b: (b, 0, 0)),
        out_shape=jax.ShapeDtypeStruct((B, N, K), _I32),
        compiler_params=cp,
    )(pos, posT)

    xf = x.reshape(B * N, DIM)
    q, voff = pl.pallas_call(
        _proj_body,
        grid=(4,),
        in_specs=[
            pl.BlockSpec((B * N // 4, DIM), lambda i: (i, 0)),
            pl.BlockSpec((DIM, DIM), lambda i: (0, 0)),
            pl.BlockSpec((DIM, DIM), lambda i: (0, 0)),
        ],
        out_specs=[
            pl.BlockSpec((B * N // 4, DIM), lambda i: (i, 0)),
            pl.BlockSpec((B * N // 4, DIM), lambda i: (i, 0)),
        ],
        out_shape=[jax.ShapeDtypeStruct((B * N, DIM), _F32)] * 2,
        compiler_params=cp,
    )(xf, Wq.T, Wvoff.T)
    q = q.reshape(B, N, DIM)
    voff = voff.reshape(B, N, DIM)

    qg = jnp.swapaxes(q.reshape(B, N, NGROUP, GD), 1, 2)
    voffg = jnp.swapaxes(voff.reshape(B, N, NGROUP, GD), 1, 2)
    xg = jnp.swapaxes(x.reshape(B, N, NGROUP, GD), 1, 2)

    cp3 = pltpu.CompilerParams(
        dimension_semantics=("parallel", "parallel", "parallel"))
    interp = pl.pallas_call(
        _offmlp_body,
        grid=(B, NGROUP, NTB),
        in_specs=[
            pl.BlockSpec((1, TB, K), lambda b, g, t: (b, t, 0)),
            pl.BlockSpec((1, 1, TB, GD), lambda b, g, t: (b, g, t, 0)),
            pl.BlockSpec((1, 1, N, GD), lambda b, g, t: (b, g, 0, 0)),
            pl.BlockSpec((1, N, 3), lambda b, g, t: (b, 0, 0)),
            pl.BlockSpec((1, 3, N), lambda b, g, t: (b, 0, 0)),
            pl.BlockSpec((1, 1, N, GD), lambda b, g, t: (b, g, 0, 0)),
            pl.BlockSpec((DIM, DIM), lambda b, g, t: (0, 0)),
            pl.BlockSpec((1, DIM), lambda b, g, t: (0, 0)),
            pl.BlockSpec((1, DIM), lambda b, g, t: (0, 0)),
            pl.BlockSpec((1, DIM), lambda b, g, t: (0, 0)),
            pl.BlockSpec((DIM, 3), lambda b, g, t: (0, 0)),
        ],
        out_specs=pl.BlockSpec((1, 1, RB, GD), lambda b, g, t: (b, g, t, 0)),
        out_shape=jax.ShapeDtypeStruct((B, NGROUP, N * K, GD), _F32),
        compiler_params=cp3,
    )(idx, qg, voffg, pos, posT, xg,
      Woff1.T, boff1.reshape(1, DIM), ln_g.reshape(1, DIM),
      ln_b.reshape(1, DIM), Woff2.T)

    cp2 = pltpu.CompilerParams(
        dimension_semantics=("parallel", "parallel"))
    y = pl.pallas_call(
        _attn_body,
        grid=(B, NTB),
        in_specs=[
            pl.BlockSpec((1, NGROUP, RB, GD), lambda b, t: (b, 0, t, 0)),
            pl.BlockSpec((1, TB, K), lambda b, t: (b, t, 0)),
            pl.BlockSpec((1, N, DIM), lambda b, t: (b, 0, 0)),
            pl.BlockSpec((DIM, DIM), lambda b, t: (0, 0)),
            pl.BlockSpec((DIM, DIM), lambda b, t: (0, 0)),
            pl.BlockSpec((DIM, DIM), lambda b, t: (0, 0)),
            pl.BlockSpec((1, DIM), lambda b, t: (0, 0)),
        ],
        out_specs=pl.BlockSpec((1, TB, DIM), lambda b, t: (b, t, 0)),
        out_shape=jax.ShapeDtypeStruct((B, N, DIM), _F32),
        compiler_params=cp2,
    )(interp, idx, q, Wk.T, Wv.T, Wproj.T, bproj.reshape(1, DIM))
    return y
